# baseline (speedup 1.0000x reference)
import functools

import jax
import jax.numpy as jnp
from jax import lax
from jax.experimental import pallas as pl
from jax.experimental.pallas import tpu as pltpu
from jax.experimental.pallas import tpu_sc as plsc

MARGIN_ = 0.2
NC = 2
NS = 16
NW = NC * NS
LANES = 16
CHUNK = 128


def _vsqrt(x):
    x = jnp.maximum(x, jnp.float32(1e-30))
    i = lax.bitcast_convert_type(x, jnp.int32)
    i = jnp.int32(0x1FBD1DF5) + lax.shift_right_logical(i, jnp.int32(1))
    y = lax.bitcast_convert_type(i, jnp.float32)
    for _ in range(3):
        y = jnp.float32(0.5) * (y + x / y)
    return y


def _make_sc_call(t_total, n_rows, d_model):
    tpw = t_total // NW
    nch = tpw // CHUNK
    mesh = plsc.VectorSubcoreMesh(core_axis_name="c", subcore_axis_name="s")

    @functools.partial(
        pl.kernel,
        mesh=mesh,
        out_type=jax.ShapeDtypeStruct((NW, 3, LANES), jnp.float32),
        scratch_types=[
            pltpu.VMEM((nch, 3 * CHUNK), jnp.int32),
            pltpu.VMEM((nch, CHUNK), jnp.int32),
            pltpu.VMEM((nch, CHUNK), jnp.int32),
            pltpu.VMEM((nch, CHUNK), jnp.int32),
            pltpu.VMEM((2, CHUNK, d_model), jnp.float32),
            pltpu.VMEM((2, CHUNK, d_model), jnp.float32),
            pltpu.VMEM((2, CHUNK, d_model), jnp.float32),
            pltpu.VMEM((3, LANES), jnp.float32),
            pltpu.SemaphoreType.DMA,
            pltpu.SemaphoreType.DMA,
        ],
        compiler_params=pltpu.CompilerParams(
            needs_layout_passes=False, use_tc_tiling_on_sc=False
        ),
    )
    def sc_fn(emb, tri, out, tri_v, idx_a, idx_p, idx_n,
              buf_a, buf_p, buf_n, res, sem0, sem1):
        sems = (sem0, sem1)
        wid = lax.axis_index("s") * NC + lax.axis_index("c")
        pltpu.sync_copy(tri.at[pl.ds(wid * nch, nch)], tri_v)

        lane = lax.iota(jnp.int32, LANES)

        def unpack(j, _):
            for g in range(CHUNK // LANES):
                col3 = (g * LANES + lane) * 3
                for c, idx in ((0, idx_a), (1, idx_p), (2, idx_n)):
                    idx[j, pl.ds(g * LANES, LANES)] = plsc.load_gather(
                        tri_v, [jnp.full((LANES,), j, jnp.int32), col3 + c])
            return 0

        lax.fori_loop(0, nch, unpack, 0)

        def start(j, b):
            pltpu.async_copy(emb.at[idx_a.at[j]], buf_a.at[b], sems[b])
            pltpu.async_copy(emb.at[idx_p.at[j]], buf_p.at[b], sems[b])
            pltpu.async_copy(emb.at[idx_n.at[j]], buf_n.at[b], sems[b])

        def wait(j, b):
            pltpu.make_async_copy(emb.at[idx_a.at[j]], buf_a.at[b], sems[b]).wait()
            pltpu.make_async_copy(emb.at[idx_p.at[j]], buf_p.at[b], sems[b]).wait()
            pltpu.make_async_copy(emb.at[idx_n.at[j]], buf_n.at[b], sems[b]).wait()

        def compute(b, accs):
            def gbody(g, accs):
                acc_l, acc_p, acc_n = accs
                rid = g * LANES + lane
                s_ap = jnp.zeros((LANES,), jnp.float32)
                s_an = jnp.zeros((LANES,), jnp.float32)
                for d in range(d_model):
                    cid = (lane + d) & (d_model - 1)
                    av = plsc.load_gather(buf_a.at[b], [rid, cid])
                    pv = plsc.load_gather(buf_p.at[b], [rid, cid])
                    nv = plsc.load_gather(buf_n.at[b], [rid, cid])
                    dp = av - pv
                    dn = av - nv
                    s_ap = s_ap + dp * dp
                    s_an = s_an + dn * dn
                dap = _vsqrt(s_ap)
                dan = _vsqrt(s_an)
                loss = jnp.maximum(dap - dan + jnp.float32(MARGIN_), 0.0)
                return (acc_l + loss, acc_p + dap, acc_n + dan)

            return lax.fori_loop(0, CHUNK // LANES, gbody, accs)

        start(0, 0)
        start(1, 1)
        zero = jnp.zeros((LANES,), jnp.float32)

        def pair(i, accs):
            for b in range(2):
                j = 2 * i + b
                wait(j, b)
                accs = compute(b, accs)

                @pl.when(j + 2 < nch)
                def _():
                    start(j + 2, b)
            return accs

        acc_l, acc_p, acc_n = lax.fori_loop(0, nch // 2, pair,
                                            (zero, zero, zero))
        res[0] = acc_l
        res[1] = acc_p
        res[2] = acc_n
        pltpu.sync_copy(res, out.at[wid])

    return sc_fn


def kernel(embeddings, target, triplets):
    del target
    t_total = triplets.shape[0]
    n_rows, d_model = embeddings.shape
    tri2 = triplets.astype(jnp.int32).reshape(t_total // CHUNK, 3 * CHUNK)
    partials = _make_sc_call(t_total, n_rows, d_model)(embeddings, tri2)
    sums = jnp.sum(partials, axis=(0, 2))
    t = jnp.float32(t_total)
    return (sums[0] / t, t_total, sums[1] / t, sums[2] / t)


# speedup vs baseline: 1.6811x; 1.6811x over previous
import functools

import jax
import jax.numpy as jnp
from jax import lax
from jax.experimental import pallas as pl
from jax.experimental.pallas import tpu as pltpu
from jax.experimental.pallas import tpu_sc as plsc

MARGIN_ = 0.2
NC = 2
NS = 16
NW = NC * NS
LANES = 16
CHUNK = 128


def _vsqrt(x):
    x = jnp.maximum(x, jnp.float32(1e-30))
    i = lax.bitcast_convert_type(x, jnp.int32)
    i = jnp.int32(0x1FBD1DF5) + lax.shift_right_logical(i, jnp.int32(1))
    y = lax.bitcast_convert_type(i, jnp.float32)
    for _ in range(3):
        y = jnp.float32(0.5) * (y + x / y)
    return y


def _make_sc_call(t_total, n_rows, d_model):
    tpw = t_total // NW
    nch = tpw // CHUNK
    mesh = plsc.VectorSubcoreMesh(core_axis_name="c", subcore_axis_name="s")

    @functools.partial(
        pl.kernel,
        mesh=mesh,
        out_type=jax.ShapeDtypeStruct((NW, 3, LANES), jnp.float32),
        scratch_types=[
            pltpu.VMEM((tpw,), jnp.int32),
            pltpu.VMEM((tpw,), jnp.int32),
            pltpu.VMEM((tpw,), jnp.int32),
            pltpu.VMEM((2, CHUNK, d_model), jnp.float32),
            pltpu.VMEM((2, CHUNK, d_model), jnp.float32),
            pltpu.VMEM((2, CHUNK, d_model), jnp.float32),
            pltpu.VMEM((3, LANES), jnp.float32),
            pltpu.SemaphoreType.DMA,
            pltpu.SemaphoreType.DMA,
        ],
        compiler_params=pltpu.CompilerParams(
            needs_layout_passes=False, use_tc_tiling_on_sc=False
        ),
    )
    def sc_fn(emb, tri3, out, idx_a, idx_p, idx_n,
              buf_a, buf_p, buf_n, res, sem0, sem1):
        sems = (sem0, sem1)
        wid = lax.axis_index("s") * NC + lax.axis_index("c")
        pltpu.sync_copy(tri3.at[0, pl.ds(wid * tpw, tpw)], idx_a)
        pltpu.sync_copy(tri3.at[1, pl.ds(wid * tpw, tpw)], idx_p)
        pltpu.sync_copy(tri3.at[2, pl.ds(wid * tpw, tpw)], idx_n)

        lane = lax.iota(jnp.int32, LANES)

        def start(j, b):
            pltpu.async_copy(emb.at[idx_a.at[pl.ds(j * CHUNK, CHUNK)]], buf_a.at[b], sems[b])
            pltpu.async_copy(emb.at[idx_p.at[pl.ds(j * CHUNK, CHUNK)]], buf_p.at[b], sems[b])
            pltpu.async_copy(emb.at[idx_n.at[pl.ds(j * CHUNK, CHUNK)]], buf_n.at[b], sems[b])

        def wait(j, b):
            pltpu.make_async_copy(emb.at[idx_a.at[pl.ds(j * CHUNK, CHUNK)]], buf_a.at[b], sems[b]).wait()
            pltpu.make_async_copy(emb.at[idx_p.at[pl.ds(j * CHUNK, CHUNK)]], buf_p.at[b], sems[b]).wait()
            pltpu.make_async_copy(emb.at[idx_n.at[pl.ds(j * CHUNK, CHUNK)]], buf_n.at[b], sems[b]).wait()

        def compute(b, accs):
            def gbody(g, accs):
                acc_l, acc_p, acc_n = accs
                rid = g * LANES + lane
                s_ap = jnp.zeros((LANES,), jnp.float32)
                s_an = jnp.zeros((LANES,), jnp.float32)
                for d in range(d_model):
                    cid = (lane + d) & (d_model - 1)
                    av = plsc.load_gather(buf_a.at[b], [rid, cid])
                    pv = plsc.load_gather(buf_p.at[b], [rid, cid])
                    nv = plsc.load_gather(buf_n.at[b], [rid, cid])
                    dp = av - pv
                    dn = av - nv
                    s_ap = s_ap + dp * dp
                    s_an = s_an + dn * dn
                dap = _vsqrt(s_ap)
                dan = _vsqrt(s_an)
                loss = jnp.maximum(dap - dan + jnp.float32(MARGIN_), 0.0)
                return (acc_l + loss, acc_p + dap, acc_n + dan)

            return lax.fori_loop(0, CHUNK // LANES, gbody, accs)

        start(0, 0)
        start(1, 1)
        zero = jnp.zeros((LANES,), jnp.float32)

        def pair(i, accs):
            for b in range(2):
                j = 2 * i + b
                wait(j, b)
                accs = compute(b, accs)

                @pl.when(j + 2 < nch)
                def _():
                    start(j + 2, b)
            return accs

        acc_l, acc_p, acc_n = lax.fori_loop(0, nch // 2, pair,
                                            (zero, zero, zero))
        res[0] = acc_l
        res[1] = acc_p
        res[2] = acc_n
        pltpu.sync_copy(res, out.at[wid])

    return sc_fn


def kernel(embeddings, target, triplets):
    del target
    t_total = triplets.shape[0]
    n_rows, d_model = embeddings.shape
    tri3 = triplets.astype(jnp.int32).T
    partials = _make_sc_call(t_total, n_rows, d_model)(embeddings, tri3)
    sums = jnp.sum(partials, axis=(0, 2))
    t = jnp.float32(t_total)
    return (sums[0] / t, t_total, sums[1] / t, sums[2] / t)
